# Initial kernel scaffold; baseline (speedup 1.0000x reference)
import jax
import jax.numpy as jnp
from jax.experimental import pallas as pl


def kernel(embedding, logits, W_route, b_route, W_noise, b_noise):
    raise NotImplementedError("write your pallas kernel here")


# speedup vs baseline: 2.8773x; 2.8773x over previous
import functools

import jax
import jax.numpy as jnp
from jax.experimental import pallas as pl


def _router_body(emb_ref, w_ref, b_ref, eps_ref, p_ref, u_ref):
    E = eps_ref.shape[-1]
    rl = jax.lax.dot_general(
        emb_ref[...], w_ref[...],
        dimension_numbers=(((1,), (1,)), ((), ())),
        preferred_element_type=jnp.float32,
    ) + b_ref[...]
    route = rl[:, :E]
    noise = rl[:, E:]
    noisy = route + eps_ref[...] * jax.nn.softplus(noise)
    mx = jnp.max(noisy, axis=-1, keepdims=True)
    ex = jnp.exp(noisy - mx)
    p = ex / jnp.sum(ex, axis=-1, keepdims=True)

    lane = jax.lax.broadcasted_iota(jnp.int32, (1, E), 1)
    rank = jnp.zeros(p.shape, jnp.int32)
    for i in range(E):
        pi = p[:, i:i + 1]
        before = (pi > p) | ((pi == p) & (i < lane))
        rank = rank + before.astype(jnp.int32)
    c = jnp.zeros(p.shape, jnp.float32)
    for i in range(E):
        c = c + p[:, i:i + 1] * (rank[:, i:i + 1] <= rank).astype(jnp.float32)
    m = (c < 0.5) | (rank == 0)
    tn = jnp.sum(m.astype(jnp.int32), axis=-1, keepdims=True)
    max_k = jnp.max(tn)
    sel = (rank < max_k).astype(p.dtype)
    tw = p * sel
    u = tw / (jnp.sum(tw, axis=-1, keepdims=True) + 1e-6)
    p_ref[...] = p
    u_ref[...] = u


def _combine_body(logits_ref, u_ref, out_ref):
    E = logits_ref.shape[0]
    acc = logits_ref[0] * u_ref[:, 0:1]
    for e in range(1, E):
        acc = acc + logits_ref[e] * u_ref[:, e:e + 1]
    out_ref[...] = acc


def kernel(embedding, logits, W_route, b_route, W_noise, b_noise):
    B, S, H = embedding.shape
    E, V = logits.shape[0], logits.shape[-1]
    T = B * S
    emb = embedding.reshape(T, H)
    w_cat = jnp.concatenate([W_route, W_noise], axis=0)
    b_cat = jnp.concatenate([b_route, b_noise]).reshape(1, 2 * E)
    eps = jax.random.normal(
        jax.random.fold_in(jax.random.key(0), 123), (B, S, E), jnp.float32
    ).reshape(T, E)

    p, u = pl.pallas_call(
        _router_body,
        out_shape=[
            jax.ShapeDtypeStruct((T, E), jnp.float32),
            jax.ShapeDtypeStruct((T, E), jnp.float32),
        ],
    )(emb, w_cat, b_cat, eps)

    Ts = 256
    while T % Ts:
        Ts //= 2
    lg = logits.reshape(E, T, V)
    combined = pl.pallas_call(
        _combine_body,
        grid=(T // Ts,),
        in_specs=[
            pl.BlockSpec((E, Ts, V), lambda i: (0, i, 0)),
            pl.BlockSpec((Ts, E), lambda i: (i, 0)),
        ],
        out_specs=pl.BlockSpec((Ts, V), lambda i: (i, 0)),
        out_shape=jax.ShapeDtypeStruct((T, V), jnp.float32),
    )(lg, u)

    route_prob = p.reshape(B, S, E)
    return combined.reshape(B, S, V), route_prob, route_prob
